# baseline (speedup 1.0000x reference)
import functools

import jax
import jax.numpy as jnp
from jax import lax
from jax.experimental import pallas as pl
from jax.experimental.pallas import tpu as pltpu
from jax.experimental.pallas import tpu_sc as plsc

NUM_NODES = 1000000
EMBED_DIM = 32
NUM_ENT = 16384

NC = 2
NS = 16
NW = NC * NS
B_PER_W = NUM_ENT // NW
CHUNK = 128
N_CHUNKS = B_PER_W // CHUNK


def _encode_body(nodes_hbm, feat_hbm, pos_hbm, out_hbm,
                 idx_v, feat_bufs, pos_bufs, out_t, sem_f, sem_p):
    wid = lax.axis_index("s") * NC + lax.axis_index("c")
    base = wid * B_PER_W

    pltpu.sync_copy(nodes_hbm.at[pl.ds(wid * N_CHUNKS, N_CHUNKS)], idx_v)

    copies = []
    for c in range(N_CHUNKS):
        copies.append(pltpu.async_copy(
            feat_hbm.at[idx_v.at[c]], feat_bufs[c], sem_f))
        copies.append(pltpu.async_copy(
            pos_hbm.at[idx_v.at[c]], pos_bufs[c], sem_p))
    for cp in copies:
        cp.wait()

    iota16 = lax.iota(jnp.int32, 16)
    for c in range(N_CHUNKS):
        fb, pb = feat_bufs[c], pos_bufs[c]

        def trans_body(i, carry, fb=fb, pb=pb, c=c):
            d = i >> 3
            e16 = i & 7
            row_idx = iota16 + (e16 << 4)
            col_idx = jnp.full((16,), 0, jnp.int32) + d
            gf = plsc.load_gather(fb, [row_idx, col_idx])
            gp = plsc.load_gather(pb, [row_idx, col_idx])
            out_t[d, pl.ds(c * CHUNK + e16 * 16, 16)] = gf + gp
            return carry

        lax.fori_loop(0, EMBED_DIM * (CHUNK // 16), trans_body, 0, unroll=4)

    pltpu.sync_copy(out_t, out_hbm.at[:, pl.ds(base, B_PER_W)])


@functools.partial(
    pl.kernel,
    mesh=plsc.VectorSubcoreMesh(core_axis_name="c", subcore_axis_name="s"),
    out_type=jax.ShapeDtypeStruct((EMBED_DIM, NUM_ENT), jnp.float32),
    scratch_types=[
        pltpu.VMEM((N_CHUNKS, CHUNK), jnp.int32),
        [pltpu.VMEM((CHUNK, EMBED_DIM), jnp.float32) for _ in range(N_CHUNKS)],
        [pltpu.VMEM((CHUNK, EMBED_DIM), jnp.float32) for _ in range(N_CHUNKS)],
        pltpu.VMEM((EMBED_DIM, B_PER_W), jnp.float32),
        pltpu.SemaphoreType.DMA,
        pltpu.SemaphoreType.DMA,
    ],
    compiler_params=pltpu.CompilerParams(
        use_tc_tiling_on_sc=False, needs_layout_passes=False),
)
def _encode(nodes_hbm, feat_hbm, pos_hbm, out_hbm,
            idx_v, feat_bufs, pos_bufs, out_t, sem_f, sem_p):
    _encode_body(nodes_hbm, feat_hbm, pos_hbm, out_hbm,
                 idx_v, feat_bufs, pos_bufs, out_t, sem_f, sem_p)


def kernel(nodes, feat_table, pos_table):
    nodes2d = nodes.astype(jnp.int32).reshape(NW * N_CHUNKS, CHUNK)
    return _encode(nodes2d, feat_table, pos_table)
